# baseline (speedup 1.0000x reference)
import functools

import jax
import jax.numpy as jnp
from jax import lax
from jax.experimental import pallas as pl
from jax.experimental.pallas import tpu as pltpu
from jax.experimental.pallas import tpu_sc as plsc

N_EMB = 1024
D_EMB = 256
COMMIT_BETA = 0.25
TOTAL_M = 16 * 32 * 32
BLOCK_M = 512
N_BLOCKS = TOTAL_M // BLOCK_M
SC_CHUNK = 128
SC_NBUF = 3


def _tc_distance_argmin_body(x_ref, e_ref, idx_ref, loss_ref, et_ref, c_ref):
    i = pl.program_id(0)
    e = e_ref[...]

    @pl.when(i == 0)
    def _once():
        c_ref[...] = jnp.sum(e * e, axis=0, keepdims=True)
        et_ref[...] = e.T

    x = x_ref[...]
    rs = jnp.sum(x * x, axis=1, keepdims=True)
    mm2 = lax.dot_general(x * -2.0, e, (((1,), (0,)), ((), ())),
                          preferred_element_type=jnp.float32)
    d = (rs + c_ref[...]) + mm2
    dmin = jnp.min(d, axis=1, keepdims=True)
    mask = jnp.where(d == dmin, 1.0, 0.0)

    row2 = lax.broadcasted_iota(jnp.int32, (2, N_EMB), 0)
    jcol = lax.broadcasted_iota(jnp.int32, (2, N_EMB), 1).astype(jnp.float32)
    w = jnp.where(row2 == 0, jcol, 1.0)
    sel = lax.dot_general(w, mask, (((1,), (1,)), ((), ())),
                          precision=lax.Precision.HIGHEST,
                          preferred_element_type=jnp.float32)
    idx_f = sel[0:1, :]
    cnt = sel[1:2, :]

    def _with_ties():
        ku = lax.broadcasted_iota(jnp.int32, (N_EMB, N_EMB), 0)
        ju = lax.broadcasted_iota(jnp.int32, (N_EMB, N_EMB), 1)
        upper = jnp.where(ku <= ju, 1.0, 0.0)
        pc = lax.dot_general(mask, upper, (((1,), (0,)), ((), ())),
                             precision=lax.Precision.HIGHEST,
                             preferred_element_type=jnp.float32)
        first = mask * jnp.where(pc == 1.0, 1.0, 0.0)
        iota1 = lax.broadcasted_iota(jnp.int32, (1, N_EMB), 1).astype(jnp.float32)
        return lax.dot_general(iota1, first, (((1,), (1,)), ((), ())),
                               precision=lax.Precision.HIGHEST,
                               preferred_element_type=jnp.float32)

    idx_f = lax.cond(jnp.max(cnt) > 1.5, _with_ties, lambda: idx_f)
    idx_ref[...] = idx_f.astype(jnp.int32).reshape(1, 1, BLOCK_M)

    part = jnp.sum(dmin).reshape(1, 1)

    @pl.when(i == 0)
    def _init():
        loss_ref[...] = part

    @pl.when(i > 0)
    def _acc():
        loss_ref[...] += part


def _tc_distance_argmin(flat, embeddings):
    return pl.pallas_call(
        _tc_distance_argmin_body,
        grid=(N_BLOCKS,),
        in_specs=[
            pl.BlockSpec((BLOCK_M, D_EMB), lambda i: (i, 0)),
            pl.BlockSpec((D_EMB, N_EMB), lambda i: (0, 0)),
        ],
        out_specs=[
            pl.BlockSpec((1, 1, BLOCK_M), lambda i: (i, 0, 0)),
            pl.BlockSpec((1, 1), lambda i: (0, 0)),
            pl.BlockSpec((N_EMB, D_EMB), lambda i: (0, 0)),
        ],
        out_shape=[
            jax.ShapeDtypeStruct((N_BLOCKS, 1, BLOCK_M), jnp.int32),
            jax.ShapeDtypeStruct((1, 1), jnp.float32),
            jax.ShapeDtypeStruct((N_EMB, D_EMB), jnp.float32),
        ],
        scratch_shapes=[pltpu.VMEM((1, N_EMB), jnp.float32)],
    )(flat, embeddings)


def _sc_gather_rows(table, idx_flat):
    info = plsc.get_sparse_core_info()
    n_workers = info.num_cores * info.num_subcores
    b_per_w = TOTAL_M // n_workers
    n_chunks = b_per_w // SC_CHUNK
    mesh = plsc.VectorSubcoreMesh(core_axis_name="c", subcore_axis_name="s")

    @functools.partial(
        pl.kernel, mesh=mesh,
        out_type=jax.ShapeDtypeStruct((TOTAL_M, D_EMB), jnp.float32),
        scratch_types=(
            [pltpu.VMEM((b_per_w,), jnp.int32)]
            + [pltpu.VMEM((SC_CHUNK, D_EMB), jnp.float32)] * SC_NBUF
            + [pltpu.SemaphoreType.DMA] * (2 * SC_NBUF)
        ),
    )
    def gather_kernel(table_hbm, idx_hbm, out_hbm, idx_v, *scr):
        rows = scr[:SC_NBUF]
        gsem = scr[SC_NBUF:2 * SC_NBUF]
        ssem = scr[2 * SC_NBUF:]
        wid = lax.axis_index("s") * info.num_cores + lax.axis_index("c")
        base = wid * b_per_w
        pltpu.sync_copy(idx_hbm.at[pl.ds(base, b_per_w)], idx_v)

        def gather_start(k):
            b = k % SC_NBUF
            return pltpu.async_copy(
                table_hbm.at[idx_v.at[pl.ds(k * SC_CHUNK, SC_CHUNK)]],
                rows[b], gsem[b])

        gathers = [gather_start(k) for k in range(min(SC_NBUF, n_chunks))]
        scatters = [None] * n_chunks
        for k in range(n_chunks):
            b = k % SC_NBUF
            gathers[k].wait()
            scatters[k] = pltpu.async_copy(
                rows[b], out_hbm.at[pl.ds(base + k * SC_CHUNK, SC_CHUNK)],
                ssem[b])
            nk = k + SC_NBUF
            if nk < n_chunks:
                scatters[nk - SC_NBUF].wait()
                gathers.append(gather_start(nk))
        for k in range(max(0, n_chunks - SC_NBUF), n_chunks):
            scatters[k].wait()

    return gather_kernel(table, idx_flat)


def kernel(inputs, embeddings):
    flat = inputs.reshape(TOTAL_M, D_EMB)
    idx3d, loss_sum, emb_t = _tc_distance_argmin(flat, embeddings)
    out_flat = _sc_gather_rows(emb_t, idx3d.reshape(TOTAL_M))
    out = out_flat.reshape(inputs.shape)
    aux_loss = (1.0 + COMMIT_BETA) * (loss_sum[0, 0] / (TOTAL_M * D_EMB))
    return out, aux_loss


# speedup vs baseline: 1.7424x; 1.7424x over previous
import functools

import jax
import jax.numpy as jnp
from jax import lax
from jax.experimental import pallas as pl
from jax.experimental.pallas import tpu as pltpu
from jax.experimental.pallas import tpu_sc as plsc

N_EMB = 1024
D_EMB = 256
COMMIT_BETA = 0.25
TOTAL_M = 16 * 32 * 32
BLOCK_M = 512
N_BLOCKS = TOTAL_M // BLOCK_M
SC_CHUNK = 128
SC_NBUF = 3


def _tc_distance_argmin_body(x_ref, e_ref, idx_ref, loss_ref, et_ref, c_ref):
    i = pl.program_id(0)
    e = e_ref[...]

    @pl.when(i == 0)
    def _once():
        c_ref[...] = jnp.sum(e * e, axis=0, keepdims=True)
        et_ref[...] = e.T

    x = x_ref[...]
    rs = jnp.sum(x * x, axis=1, keepdims=True)
    mm2 = lax.dot_general(x * -2.0, e, (((1,), (0,)), ((), ())),
                          preferred_element_type=jnp.float32)
    d = (rs + c_ref[...]) + mm2
    dmin = jnp.min(d, axis=1, keepdims=True)
    iota = lax.broadcasted_iota(jnp.int32, d.shape, 1).astype(jnp.float32)
    key = jnp.where(d == dmin, iota, jnp.float32(N_EMB))
    idx_ref[...] = jnp.min(key, axis=1, keepdims=True).astype(jnp.int32)

    part = jnp.sum(dmin).reshape(1, 1)

    @pl.when(i == 0)
    def _init():
        loss_ref[...] = part

    @pl.when(i > 0)
    def _acc():
        loss_ref[...] += part


def _tc_distance_argmin(flat, embeddings):
    return pl.pallas_call(
        _tc_distance_argmin_body,
        grid=(N_BLOCKS,),
        in_specs=[
            pl.BlockSpec((BLOCK_M, D_EMB), lambda i: (i, 0)),
            pl.BlockSpec((D_EMB, N_EMB), lambda i: (0, 0)),
        ],
        out_specs=[
            pl.BlockSpec((BLOCK_M, 1), lambda i: (i, 0)),
            pl.BlockSpec((1, 1), lambda i: (0, 0)),
            pl.BlockSpec((N_EMB, D_EMB), lambda i: (0, 0)),
        ],
        out_shape=[
            jax.ShapeDtypeStruct((TOTAL_M, 1), jnp.int32),
            jax.ShapeDtypeStruct((1, 1), jnp.float32),
            jax.ShapeDtypeStruct((N_EMB, D_EMB), jnp.float32),
        ],
        scratch_shapes=[pltpu.VMEM((1, N_EMB), jnp.float32)],
    )(flat, embeddings)


def _sc_gather_rows(table, idx_flat):
    info = plsc.get_sparse_core_info()
    n_workers = info.num_cores * info.num_subcores
    b_per_w = TOTAL_M // n_workers
    n_chunks = b_per_w // SC_CHUNK
    mesh = plsc.VectorSubcoreMesh(core_axis_name="c", subcore_axis_name="s")

    @functools.partial(
        pl.kernel, mesh=mesh,
        out_type=jax.ShapeDtypeStruct((TOTAL_M, D_EMB), jnp.float32),
        scratch_types=(
            [pltpu.VMEM((b_per_w,), jnp.int32)]
            + [pltpu.VMEM((SC_CHUNK, D_EMB), jnp.float32)] * SC_NBUF
            + [pltpu.SemaphoreType.DMA] * (2 * SC_NBUF)
        ),
    )
    def gather_kernel(table_hbm, idx_hbm, out_hbm, idx_v, *scr):
        rows = scr[:SC_NBUF]
        gsem = scr[SC_NBUF:2 * SC_NBUF]
        ssem = scr[2 * SC_NBUF:]
        wid = lax.axis_index("s") * info.num_cores + lax.axis_index("c")
        base = wid * b_per_w
        pltpu.sync_copy(idx_hbm.at[pl.ds(base, b_per_w)], idx_v)

        def gather_start(k):
            b = k % SC_NBUF
            return pltpu.async_copy(
                table_hbm.at[idx_v.at[pl.ds(k * SC_CHUNK, SC_CHUNK)]],
                rows[b], gsem[b])

        gathers = [gather_start(k) for k in range(min(SC_NBUF, n_chunks))]
        scatters = [None] * n_chunks
        for k in range(n_chunks):
            b = k % SC_NBUF
            gathers[k].wait()
            scatters[k] = pltpu.async_copy(
                rows[b], out_hbm.at[pl.ds(base + k * SC_CHUNK, SC_CHUNK)],
                ssem[b])
            nk = k + SC_NBUF
            if nk < n_chunks:
                scatters[nk - SC_NBUF].wait()
                gathers.append(gather_start(nk))
        for k in range(max(0, n_chunks - SC_NBUF), n_chunks):
            scatters[k].wait()

    return gather_kernel(table, idx_flat)


def kernel(inputs, embeddings):
    flat = inputs.reshape(TOTAL_M, D_EMB)
    idx2d, loss_sum, emb_t = _tc_distance_argmin(flat, embeddings)
    out_flat = _sc_gather_rows(emb_t, idx2d.reshape(TOTAL_M))
    out = out_flat.reshape(inputs.shape)
    aux_loss = (1.0 + COMMIT_BETA) * (loss_sum[0, 0] / (TOTAL_M * D_EMB))
    return out, aux_loss


# speedup vs baseline: 1.8388x; 1.0553x over previous
import functools

import jax
import jax.numpy as jnp
from jax import lax
from jax.experimental import pallas as pl
from jax.experimental.pallas import tpu as pltpu
from jax.experimental.pallas import tpu_sc as plsc

N_EMB = 1024
D_EMB = 256
COMMIT_BETA = 0.25
TOTAL_M = 16 * 32 * 32
BLOCK_M = 512
N_BLOCKS = TOTAL_M // BLOCK_M
SC_CHUNK = 128
SC_NBUF = 3


def _tc_distance_argmin_body(x_ref, e_ref, idx_ref, loss_ref, et_ref, ct_ref):
    i = pl.program_id(0)

    @pl.when(i == 0)
    def _once():
        et = e_ref[...].T
        et_ref[...] = et
        ct_ref[...] = jnp.sum(et * et, axis=1, keepdims=True)

    x = x_ref[...]
    ones_row = jnp.ones((1, D_EMB), jnp.float32)
    rst = lax.dot_general(ones_row, x * x, (((1,), (1,)), ((), ())),
                          preferred_element_type=jnp.float32)
    mm2 = lax.dot_general(et_ref[...], x * -2.0, (((1,), (1,)), ((), ())),
                          preferred_element_type=jnp.float32)
    d = (ct_ref[...] + rst) + mm2
    dmin = jnp.min(d, axis=0, keepdims=True)
    iota = lax.broadcasted_iota(jnp.int32, d.shape, 0).astype(jnp.float32)
    key = jnp.where(d == dmin, iota, jnp.float32(N_EMB))
    idx_ref[...] = jnp.min(key, axis=0, keepdims=True).astype(
        jnp.int32).reshape(1, 1, BLOCK_M)

    part = jnp.sum(dmin).reshape(1, 1)

    @pl.when(i == 0)
    def _init():
        loss_ref[...] = part

    @pl.when(i > 0)
    def _acc():
        loss_ref[...] += part


def _tc_distance_argmin(flat, embeddings):
    return pl.pallas_call(
        _tc_distance_argmin_body,
        grid=(N_BLOCKS,),
        in_specs=[
            pl.BlockSpec((BLOCK_M, D_EMB), lambda i: (i, 0)),
            pl.BlockSpec((D_EMB, N_EMB), lambda i: (0, 0)),
        ],
        out_specs=[
            pl.BlockSpec((1, 1, BLOCK_M), lambda i: (i, 0, 0)),
            pl.BlockSpec((1, 1), lambda i: (0, 0)),
            pl.BlockSpec((N_EMB, D_EMB), lambda i: (0, 0)),
        ],
        out_shape=[
            jax.ShapeDtypeStruct((N_BLOCKS, 1, BLOCK_M), jnp.int32),
            jax.ShapeDtypeStruct((1, 1), jnp.float32),
            jax.ShapeDtypeStruct((N_EMB, D_EMB), jnp.float32),
        ],
        scratch_shapes=[pltpu.VMEM((N_EMB, 1), jnp.float32)],
    )(flat, embeddings)


def _sc_gather_rows(table, idx2d):
    info = plsc.get_sparse_core_info()
    n_workers = info.num_cores * info.num_subcores
    b_per_w = TOTAL_M // n_workers
    n_chunks = b_per_w // SC_CHUNK
    mesh = plsc.VectorSubcoreMesh(core_axis_name="c", subcore_axis_name="s")

    @functools.partial(
        pl.kernel, mesh=mesh,
        out_type=jax.ShapeDtypeStruct((TOTAL_M, D_EMB), jnp.float32),
        scratch_types=(
            [pltpu.VMEM((b_per_w,), jnp.int32)]
            + [pltpu.VMEM((SC_CHUNK, D_EMB), jnp.float32)] * SC_NBUF
            + [pltpu.SemaphoreType.DMA] * (2 * SC_NBUF)
        ),
    )
    def gather_kernel(table_hbm, idx_hbm, out_hbm, idx_v, *scr):
        rows = scr[:SC_NBUF]
        gsem = scr[SC_NBUF:2 * SC_NBUF]
        ssem = scr[2 * SC_NBUF:]
        wid = lax.axis_index("s") * info.num_cores + lax.axis_index("c")
        base = wid * b_per_w
        row = base // BLOCK_M
        col = base % BLOCK_M
        pltpu.sync_copy(idx_hbm.at[row, 0, pl.ds(col, b_per_w)], idx_v)

        def gather_start(k):
            b = k % SC_NBUF
            return pltpu.async_copy(
                table_hbm.at[idx_v.at[pl.ds(k * SC_CHUNK, SC_CHUNK)]],
                rows[b], gsem[b])

        gathers = [gather_start(k) for k in range(min(SC_NBUF, n_chunks))]
        scatters = [None] * n_chunks
        for k in range(n_chunks):
            b = k % SC_NBUF
            gathers[k].wait()
            scatters[k] = pltpu.async_copy(
                rows[b], out_hbm.at[pl.ds(base + k * SC_CHUNK, SC_CHUNK)],
                ssem[b])
            nk = k + SC_NBUF
            if nk < n_chunks:
                scatters[k].wait()
                gathers.append(gather_start(nk))
        for k in range(max(0, n_chunks - SC_NBUF), n_chunks):
            scatters[k].wait()

    return gather_kernel(table, idx2d)


def kernel(inputs, embeddings):
    flat = inputs.reshape(TOTAL_M, D_EMB)
    idx2d, loss_sum, emb_t = _tc_distance_argmin(flat, embeddings)
    out_flat = _sc_gather_rows(emb_t, idx2d)
    out = out_flat.reshape(inputs.shape)
    aux_loss = (1.0 + COMMIT_BETA) * (loss_sum[0, 0] / (TOTAL_M * D_EMB))
    return out, aux_loss


# speedup vs baseline: 1.9861x; 1.0801x over previous
import functools

import jax
import jax.numpy as jnp
from jax import lax
from jax.experimental import pallas as pl
from jax.experimental.pallas import tpu as pltpu
from jax.experimental.pallas import tpu_sc as plsc

N_EMB = 1024
D_EMB = 256
COMMIT_BETA = 0.25
TOTAL_M = 16 * 32 * 32
BLOCK_M = 1024
N_BLOCKS = TOTAL_M // BLOCK_M
SC_CHUNK = 128
SC_NBUF = 3


def _tc_distance_argmin_body(x_ref, e_ref, idx_ref, loss_ref, et_ref, ct_ref):
    i = pl.program_id(0)

    @pl.when(i == 0)
    def _once():
        et = e_ref[...].T
        et_ref[...] = et
        ct_ref[...] = jnp.sum(et * et, axis=1, keepdims=True)

    x = x_ref[...]
    ones_row = jnp.ones((1, D_EMB), jnp.float32)
    rst = lax.dot_general(ones_row, x * x, (((1,), (1,)), ((), ())),
                          preferred_element_type=jnp.float32)
    mm2 = lax.dot_general(et_ref[...], x * -2.0, (((1,), (1,)), ((), ())),
                          preferred_element_type=jnp.float32)
    d = (ct_ref[...] + rst) + mm2
    dmin = jnp.min(d, axis=0, keepdims=True)
    iota = lax.broadcasted_iota(jnp.int32, d.shape, 0).astype(jnp.float32)
    key = jnp.where(d == dmin, iota, jnp.float32(N_EMB))
    idx_ref[...] = jnp.min(key, axis=0, keepdims=True).astype(
        jnp.int32).reshape(1, 1, BLOCK_M)

    part = jnp.sum(dmin).reshape(1, 1)

    @pl.when(i == 0)
    def _init():
        loss_ref[...] = part

    @pl.when(i > 0)
    def _acc():
        loss_ref[...] += part


def _tc_distance_argmin(flat, embeddings):
    return pl.pallas_call(
        _tc_distance_argmin_body,
        grid=(N_BLOCKS,),
        in_specs=[
            pl.BlockSpec((BLOCK_M, D_EMB), lambda i: (i, 0)),
            pl.BlockSpec((D_EMB, N_EMB), lambda i: (0, 0)),
        ],
        out_specs=[
            pl.BlockSpec((1, 1, BLOCK_M), lambda i: (i, 0, 0)),
            pl.BlockSpec((1, 1), lambda i: (0, 0)),
            pl.BlockSpec((N_EMB, D_EMB), lambda i: (0, 0)),
        ],
        out_shape=[
            jax.ShapeDtypeStruct((N_BLOCKS, 1, BLOCK_M), jnp.int32),
            jax.ShapeDtypeStruct((1, 1), jnp.float32),
            jax.ShapeDtypeStruct((N_EMB, D_EMB), jnp.float32),
        ],
        scratch_shapes=[pltpu.VMEM((N_EMB, 1), jnp.float32)],
    )(flat, embeddings)


def _sc_gather_rows(table, idx2d):
    info = plsc.get_sparse_core_info()
    n_workers = info.num_cores * info.num_subcores
    b_per_w = TOTAL_M // n_workers
    n_chunks = b_per_w // SC_CHUNK
    mesh = plsc.VectorSubcoreMesh(core_axis_name="c", subcore_axis_name="s")

    @functools.partial(
        pl.kernel, mesh=mesh,
        out_type=jax.ShapeDtypeStruct((TOTAL_M, D_EMB), jnp.float32),
        scratch_types=(
            [pltpu.VMEM((b_per_w,), jnp.int32)]
            + [pltpu.VMEM((SC_CHUNK, D_EMB), jnp.float32)] * SC_NBUF
            + [pltpu.SemaphoreType.DMA] * (2 * SC_NBUF)
        ),
    )
    def gather_kernel(table_hbm, idx_hbm, out_hbm, idx_v, *scr):
        rows = scr[:SC_NBUF]
        gsem = scr[SC_NBUF:2 * SC_NBUF]
        ssem = scr[2 * SC_NBUF:]
        wid = lax.axis_index("s") * info.num_cores + lax.axis_index("c")
        base = wid * b_per_w
        row = base // BLOCK_M
        col = base % BLOCK_M
        pltpu.sync_copy(idx_hbm.at[row, 0, pl.ds(col, b_per_w)], idx_v)

        def gather_start(k):
            b = k % SC_NBUF
            return pltpu.async_copy(
                table_hbm.at[idx_v.at[pl.ds(k * SC_CHUNK, SC_CHUNK)]],
                rows[b], gsem[b])

        gathers = [gather_start(k) for k in range(min(SC_NBUF, n_chunks))]
        scatters = [None] * n_chunks
        for k in range(n_chunks):
            b = k % SC_NBUF
            gathers[k].wait()
            scatters[k] = pltpu.async_copy(
                rows[b], out_hbm.at[pl.ds(base + k * SC_CHUNK, SC_CHUNK)],
                ssem[b])
            nk = k + SC_NBUF
            if nk < n_chunks:
                scatters[k].wait()
                gathers.append(gather_start(nk))
        for k in range(max(0, n_chunks - SC_NBUF), n_chunks):
            scatters[k].wait()

    return gather_kernel(table, idx2d)


def kernel(inputs, embeddings):
    flat = inputs.reshape(TOTAL_M, D_EMB)
    idx2d, loss_sum, emb_t = _tc_distance_argmin(flat, embeddings)
    out_flat = _sc_gather_rows(emb_t, idx2d)
    out = out_flat.reshape(inputs.shape)
    aux_loss = (1.0 + COMMIT_BETA) * (loss_sum[0, 0] / (TOTAL_M * D_EMB))
    return out, aux_loss
